# baseline (speedup 1.0000x reference)
import jax
import jax.numpy as jnp
from jax import lax
from jax.experimental import pallas as pl
from jax.experimental.pallas import tpu as pltpu

_NA = 3
_NO = 85
_H = 64
_W = 64
_HW = _H * _W
_STRIDE = 8.0


def _decode_body(p_ref, anch_ref, o_ref):
    a = pl.program_id(1)
    x = p_ref[0, 0]
    k = lax.broadcasted_iota(jnp.int32, (_NO, _HW), 0)
    j = lax.broadcasted_iota(jnp.int32, (_NO, _HW), 1)
    gx = (j & (_W - 1)).astype(jnp.float32)
    gy = (j >> 6).astype(jnp.float32)

    e = jnp.exp(-x)
    is_wh = (k == 2) | (k == 3)
    denom = jnp.where(is_wh, e, 1.0 + e)
    val = 1.0 / denom

    aw = anch_ref[a, 0]
    ah = anch_ref[a, 1]
    mult = jnp.where(k < 2, _STRIDE, jnp.where(k == 2, aw, jnp.where(k == 3, ah, 1.0)))
    add = jnp.where(k == 0, gx, jnp.where(k == 1, gy, 0.0)) * _STRIDE

    o_ref[0, 0] = val * mult + add


def kernel(preds, anchors):
    n, c, h, w = preds.shape
    p = preds.reshape(n, _NA, _NO, h * w)
    out = pl.pallas_call(
        _decode_body,
        grid=(n, _NA),
        in_specs=[
            pl.BlockSpec((1, 1, _NO, _HW), lambda i, a: (i, a, 0, 0)),
            pl.BlockSpec(memory_space=pltpu.SMEM),
        ],
        out_specs=pl.BlockSpec((1, 1, _NO, _HW), lambda i, a: (i, a, 0, 0)),
        out_shape=jax.ShapeDtypeStruct((n, _NA, _NO, _HW), jnp.float32),
        compiler_params=pltpu.CompilerParams(
            dimension_semantics=("parallel", "parallel"),
        ),
    )(p, anchors)
    return out.transpose(0, 1, 3, 2).reshape(n, _NA * _HW, _NO)


# speedup vs baseline: 2.1655x; 2.1655x over previous
import jax
import jax.numpy as jnp
from jax import lax
from jax.experimental import pallas as pl
from jax.experimental.pallas import tpu as pltpu

_NA = 3
_NO = 85
_H = 64
_W = 64
_HW = _H * _W
_STRIDE = 8.0


def _decode_body(p_ref, anch_ref, o_ref):
    x = p_ref[0]
    r = lax.broadcasted_iota(jnp.int32, (_NA * _NO, _HW), 0)
    k = r % _NO
    j = lax.broadcasted_iota(jnp.int32, (_NA * _NO, _HW), 1)
    gx = (j & (_W - 1)).astype(jnp.float32)
    gy = (j >> 6).astype(jnp.float32)

    e = jnp.exp(-x)
    is_wh = (k == 2) | (k == 3)
    denom = jnp.where(is_wh, e, 1.0 + e)
    val = 1.0 / denom

    a0 = jnp.where(r < _NO, anch_ref[0, 0], jnp.where(r < 2 * _NO, anch_ref[1, 0], anch_ref[2, 0]))
    a1 = jnp.where(r < _NO, anch_ref[0, 1], jnp.where(r < 2 * _NO, anch_ref[1, 1], anch_ref[2, 1]))
    mult = jnp.where(k < 2, _STRIDE, jnp.where(k == 2, a0, jnp.where(k == 3, a1, 1.0)))
    add = jnp.where(k == 0, gx, jnp.where(k == 1, gy, 0.0)) * _STRIDE

    o_ref[0] = val * mult + add


def kernel(preds, anchors):
    n, c, h, w = preds.shape
    p = preds.reshape(n, c, h * w)
    out = pl.pallas_call(
        _decode_body,
        grid=(n,),
        in_specs=[
            pl.BlockSpec((1, c, _HW), lambda i: (i, 0, 0)),
            pl.BlockSpec(memory_space=pltpu.SMEM),
        ],
        out_specs=pl.BlockSpec((1, c, _HW), lambda i: (i, 0, 0)),
        out_shape=jax.ShapeDtypeStruct((n, c, _HW), jnp.float32),
        compiler_params=pltpu.CompilerParams(
            dimension_semantics=("parallel",),
        ),
    )(p, anchors)
    return out.reshape(n, _NA, _NO, _HW).transpose(0, 1, 3, 2).reshape(n, _NA * _HW, _NO)


# speedup vs baseline: 2.1875x; 1.0102x over previous
import jax
import jax.numpy as jnp
from jax import lax
from jax.experimental import pallas as pl
from jax.experimental.pallas import tpu as pltpu

_NA = 3
_NO = 85
_H = 64
_W = 64
_HW = _H * _W
_STRIDE = 8.0


_NB = 2


def _decode_body(p_ref, anch_ref, o_ref):
    x = p_ref[...]
    shp = (_NB, _NA * _NO, _HW)
    r = lax.broadcasted_iota(jnp.int32, shp, 1)
    k = r - jnp.where(r < _NO, 0, jnp.where(r < 2 * _NO, _NO, 2 * _NO))
    j = lax.broadcasted_iota(jnp.int32, shp, 2)
    gx = (j & (_W - 1)).astype(jnp.float32)
    gy = (j >> 6).astype(jnp.float32)

    e = jnp.exp(-x)
    is_wh = (k == 2) | (k == 3)
    denom = jnp.where(is_wh, e, 1.0 + e)
    val = 1.0 / denom

    a0 = jnp.where(r < _NO, anch_ref[0, 0], jnp.where(r < 2 * _NO, anch_ref[1, 0], anch_ref[2, 0]))
    a1 = jnp.where(r < _NO, anch_ref[0, 1], jnp.where(r < 2 * _NO, anch_ref[1, 1], anch_ref[2, 1]))
    mult = jnp.where(k < 2, _STRIDE, jnp.where(k == 2, a0, jnp.where(k == 3, a1, 1.0)))
    add = jnp.where(k == 0, gx, jnp.where(k == 1, gy, 0.0)) * _STRIDE

    o_ref[...] = val * mult + add


def kernel(preds, anchors):
    n, c, h, w = preds.shape
    p = preds.reshape(n, c, h * w)
    out = pl.pallas_call(
        _decode_body,
        grid=(n // _NB,),
        in_specs=[
            pl.BlockSpec((_NB, c, _HW), lambda i: (i, 0, 0)),
            pl.BlockSpec(memory_space=pltpu.SMEM),
        ],
        out_specs=pl.BlockSpec((_NB, c, _HW), lambda i: (i, 0, 0)),
        out_shape=jax.ShapeDtypeStruct((n, c, _HW), jnp.float32),
        compiler_params=pltpu.CompilerParams(
            dimension_semantics=("parallel",),
        ),
    )(p, anchors)
    return out.reshape(n, _NA, _NO, _HW).transpose(0, 1, 3, 2).reshape(n, _NA * _HW, _NO)


# speedup vs baseline: 2.4710x; 1.1296x over previous
import jax
import jax.numpy as jnp
from jax import lax
from jax.experimental import pallas as pl
from jax.experimental.pallas import tpu as pltpu

_NA = 3
_NO = 85
_H = 64
_W = 64
_HW = _H * _W
_STRIDE = 8.0


_NB = 2


def _decode_body(p_ref, anch_ref, o_ref):
    x = p_ref[...]
    shp = (_NB, _NA * _NO, _HW)
    r = lax.broadcasted_iota(jnp.int32, shp, 1)
    k = r - jnp.where(r < _NO, 0, jnp.where(r < 2 * _NO, _NO, 2 * _NO))
    j = lax.broadcasted_iota(jnp.int32, shp, 2)
    gx = (j & (_W - 1)).astype(jnp.float32)
    gy = (j >> 6).astype(jnp.float32)

    e = jnp.exp(-x)
    is_wh = (k == 2) | (k == 3)
    denom = jnp.where(is_wh, e, 1.0 + e)
    val = 1.0 / denom

    a0 = jnp.where(r < _NO, anch_ref[0, 0], jnp.where(r < 2 * _NO, anch_ref[1, 0], anch_ref[2, 0]))
    a1 = jnp.where(r < _NO, anch_ref[0, 1], jnp.where(r < 2 * _NO, anch_ref[1, 1], anch_ref[2, 1]))
    mult = jnp.where(k < 2, _STRIDE, jnp.where(k == 2, a0, jnp.where(k == 3, a1, 1.0)))
    add = jnp.where(k == 0, gx, jnp.where(k == 1, gy, 0.0)) * _STRIDE

    d = val * mult + add
    o_ref[...] = jnp.concatenate(
        [jnp.swapaxes(d[:, a * _NO:(a + 1) * _NO, :], 1, 2) for a in range(_NA)],
        axis=1,
    )


def kernel(preds, anchors):
    n, c, h, w = preds.shape
    p = preds.reshape(n, c, h * w)
    out = pl.pallas_call(
        _decode_body,
        grid=(n // _NB,),
        in_specs=[
            pl.BlockSpec((_NB, c, _HW), lambda i: (i, 0, 0)),
            pl.BlockSpec(memory_space=pltpu.SMEM),
        ],
        out_specs=pl.BlockSpec((_NB, _NA * _HW, _NO), lambda i: (i, 0, 0)),
        out_shape=jax.ShapeDtypeStruct((n, _NA * _HW, _NO), jnp.float32),
        compiler_params=pltpu.CompilerParams(
            dimension_semantics=("parallel",),
        ),
    )(p, anchors)
    return out
